# Initial kernel scaffold; baseline (speedup 1.0000x reference)
import jax
import jax.numpy as jnp
from jax.experimental import pallas as pl


def kernel(sound_fragment, key):
    raise NotImplementedError("write your pallas kernel here")


# speedup vs baseline: 39.1417x; 39.1417x over previous
import functools
import jax
import jax.numpy as jnp
from jax import lax
from jax.experimental import pallas as pl
from jax.experimental.pallas import tpu as pltpu
from jax.experimental.pallas import tpu_sc as plsc

_FRAGMENT_LENGTH = 2097152
_WINDOW_SIZE = 4096
_BATCH_SIZE = 1024
_NUM_CORES = 2
_NUM_SUBCORES = 16
_NUM_WORKERS = _NUM_CORES * _NUM_SUBCORES
_B_PER_W = _BATCH_SIZE // _NUM_WORKERS
_LANES = 16
_CHUNKS = _WINDOW_SIZE // _LANES
_INV_LEN = 1.0 / _FRAGMENT_LENGTH


def _sampler_body(frag_hbm, starts_hbm, time_hbm, press_hbm,
                  starts_v, buf_v, row_v, trow_v):
    wid = lax.axis_index("c") * _NUM_SUBCORES + lax.axis_index("s")
    base = wid * _B_PER_W
    pltpu.sync_copy(starts_hbm.at[pl.ds(base, _B_PER_W)],
                    starts_v.at[pl.ds(0, _B_PER_W)])

    iotaf = lax.iota(jnp.int32, _LANES).astype(jnp.float32)

    def per_window(i, carry):
        s = starts_v[pl.ds(i, _LANES)][0]
        q = pl.multiple_of((s // 8) * 8, 8)
        r = s - q
        pltpu.sync_copy(frag_hbm.at[pl.ds(q, _WINDOW_SIZE + 8)], buf_v)

        def inner(j, c):
            off = j * _LANES
            row_v[pl.ds(off, _LANES)] = buf_v[pl.ds(r + off, _LANES)]
            tbase = (s + off).astype(jnp.float32)
            trow_v[pl.ds(off, _LANES)] = (iotaf + tbase) * _INV_LEN
            return c

        lax.fori_loop(0, _CHUNKS, inner, 0, unroll=4)
        row = base + i
        pltpu.sync_copy(row_v, press_hbm.at[row])
        pltpu.sync_copy(trow_v, time_hbm.at[row])
        return carry

    lax.fori_loop(0, _B_PER_W, per_window, 0)


@functools.partial(jax.jit)
def _sampler_call(sound_fragment, start_points):
    mesh = plsc.VectorSubcoreMesh(core_axis_name="c", subcore_axis_name="s")
    f = pl.kernel(
        _sampler_body,
        mesh=mesh,
        out_type=[
            jax.ShapeDtypeStruct((_BATCH_SIZE, _WINDOW_SIZE), jnp.float32),
            jax.ShapeDtypeStruct((_BATCH_SIZE, _WINDOW_SIZE), jnp.float32),
        ],
        scratch_types=[
            pltpu.VMEM((_B_PER_W + _LANES,), jnp.int32),
            pltpu.VMEM((_WINDOW_SIZE + 8,), jnp.float32),
            pltpu.VMEM((_WINDOW_SIZE,), jnp.float32),
            pltpu.VMEM((_WINDOW_SIZE,), jnp.float32),
        ],
    )
    return f(sound_fragment, start_points)


def kernel(sound_fragment, key):
    start_points = jax.random.uniform(
        key, shape=(_BATCH_SIZE,), minval=0,
        maxval=_FRAGMENT_LENGTH - _WINDOW_SIZE,
    )
    start_points = jnp.floor(start_points).astype(jnp.int32)
    time_points, pressure_values = _sampler_call(sound_fragment, start_points)
    return (time_points, pressure_values)


# speedup vs baseline: 68.6095x; 1.7528x over previous
import functools
import jax
import jax.numpy as jnp
from jax import lax
from jax.experimental import pallas as pl
from jax.experimental.pallas import tpu as pltpu
from jax.experimental.pallas import tpu_sc as plsc

_FRAGMENT_LENGTH = 2097152
_WINDOW_SIZE = 4096
_BATCH_SIZE = 1024
_NUM_CORES = 2
_NUM_SUBCORES = 16
_NUM_WORKERS = _NUM_CORES * _NUM_SUBCORES
_B_PER_W = _BATCH_SIZE // _NUM_WORKERS
_LANES = 16
_CHUNKS = _WINDOW_SIZE // _LANES
_INV_LEN = 1.0 / _FRAGMENT_LENGTH
_K = 4
_G = _B_PER_W // _K
_BUF_W = _WINDOW_SIZE + 8


def _press_body(frag_hbm, starts_hbm, press_hbm, starts_v, *scratch):
    bufs = scratch[:2 * _K]
    rows = scratch[2 * _K:4 * _K]
    sem_in, sem_out = scratch[4 * _K], scratch[4 * _K + 1]
    wid = lax.axis_index("c") * _NUM_SUBCORES + lax.axis_index("s")
    base = wid * _B_PER_W
    pltpu.sync_copy(starts_hbm.at[pl.ds(base, _B_PER_W)],
                    starts_v.at[pl.ds(0, _B_PER_W)])

    def fire_in(g, slot):
        cps = []
        for k in range(_K):
            w = g * _K + k
            s = starts_v[pl.ds(w, _LANES)][0]
            q = pl.multiple_of((s // 8) * 8, 8)
            r = s - q
            cp = pltpu.async_copy(frag_hbm.at[pl.ds(q, _BUF_W)],
                                  bufs[slot * _K + k], sem_in)
            cps.append((cp, r))
        return cps

    in_flight = fire_in(0, 0)
    outs = []
    for g in range(_G):
        slot = g % 2
        nxt = fire_in(g + 1, 1 - slot) if g + 1 < _G else []
        if g >= 2:
            for cp in outs[g - 2]:
                cp.wait()
        cur = []
        for k in range(_K):
            cp_in, r = in_flight[k]
            cp_in.wait()
            buf = bufs[slot * _K + k]
            row = rows[slot * _K + k]

            def inner(j, c, _buf=buf, _row=row, _r=r):
                off = j * _LANES
                _row[pl.ds(off, _LANES)] = _buf[pl.ds(_r + off, _LANES)]
                return c

            lax.fori_loop(0, _CHUNKS, inner, 0, unroll=8)
            cur.append(pltpu.async_copy(row,
                                        press_hbm.at[base + g * _K + k],
                                        sem_out))
        outs.append(cur)
        in_flight = nxt
    for cp in outs[-2]:
        cp.wait()
    for cp in outs[-1]:
        cp.wait()


def _time_body(starts_ref, out_ref):
    iota = lax.broadcasted_iota(jnp.int32, out_ref.shape, 1).astype(
        jnp.float32)
    out_ref[...] = (starts_ref[...] + iota) * _INV_LEN


_TIME_BLK = 128


@jax.jit
def _sampler_call(sound_fragment, start_points):
    mesh = plsc.VectorSubcoreMesh(core_axis_name="c", subcore_axis_name="s")
    press_fn = pl.kernel(
        _press_body,
        mesh=mesh,
        out_type=jax.ShapeDtypeStruct((_BATCH_SIZE, _WINDOW_SIZE),
                                      jnp.float32),
        scratch_types=[
            pltpu.VMEM((_B_PER_W + _LANES,), jnp.int32),
        ] + [pltpu.VMEM((_BUF_W,), jnp.float32) for _ in range(2 * _K)]
        + [pltpu.VMEM((_WINDOW_SIZE,), jnp.float32) for _ in range(2 * _K)]
        + [
            pltpu.SemaphoreType.DMA,
            pltpu.SemaphoreType.DMA,
        ],
    )
    pressure = press_fn(sound_fragment, start_points)

    starts_f = start_points.astype(jnp.float32)[:, None]
    time_points = pl.pallas_call(
        _time_body,
        grid=(_BATCH_SIZE // _TIME_BLK,),
        in_specs=[pl.BlockSpec((_TIME_BLK, 1), lambda i: (i, 0))],
        out_specs=pl.BlockSpec((_TIME_BLK, _WINDOW_SIZE), lambda i: (i, 0)),
        out_shape=jax.ShapeDtypeStruct((_BATCH_SIZE, _WINDOW_SIZE),
                                       jnp.float32),
    )(starts_f)
    return time_points, pressure


def kernel(sound_fragment, key):
    start_points = jax.random.uniform(
        key, shape=(_BATCH_SIZE,), minval=0,
        maxval=_FRAGMENT_LENGTH - _WINDOW_SIZE,
    )
    start_points = jnp.floor(start_points).astype(jnp.int32)
    time_points, pressure_values = _sampler_call(sound_fragment, start_points)
    return (time_points, pressure_values)


# speedup vs baseline: 88.6504x; 1.2921x over previous
import functools
import jax
import jax.numpy as jnp
from jax import lax
from jax.experimental import pallas as pl
from jax.experimental.pallas import tpu as pltpu
from jax.experimental.pallas import tpu_sc as plsc

_FRAGMENT_LENGTH = 2097152
_WINDOW_SIZE = 4096
_BATCH_SIZE = 1024
_NUM_CORES = 2
_NUM_SUBCORES = 16
_NUM_WORKERS = _NUM_CORES * _NUM_SUBCORES
_B_PER_W = _BATCH_SIZE // _NUM_WORKERS
_LANES = 16
_CHUNKS = _WINDOW_SIZE // _LANES
_INV_LEN = 1.0 / _FRAGMENT_LENGTH
_K = 4
_G = _B_PER_W // _K
_BUF_W = _WINDOW_SIZE + 8
_SHIFT_GRP = 16


def _press_body(frag_hbm, starts_hbm, press_hbm, starts_v, *scratch):
    bufs = scratch[:2 * _K]
    rows = scratch[2 * _K:4 * _K]
    sem_in, sem_out = scratch[4 * _K], scratch[4 * _K + 1]
    wid = lax.axis_index("c") * _NUM_SUBCORES + lax.axis_index("s")
    base = wid * _B_PER_W
    pltpu.sync_copy(starts_hbm.at[pl.ds(base, _B_PER_W)],
                    starts_v.at[pl.ds(0, _B_PER_W)])

    def fire_in(g, slot):
        cps = []
        for k in range(_K):
            w = g * _K + k
            s = starts_v[pl.ds(w, _LANES)][0]
            q = pl.multiple_of((s // 8) * 8, 8)
            r = s - q
            cp = pltpu.async_copy(frag_hbm.at[pl.ds(q, _BUF_W)],
                                  bufs[slot * _K + k], sem_in)
            cps.append((cp, r))
        return cps

    in_flight = fire_in(0, 0)
    outs = []
    for g in range(_G):
        slot = g % 2
        nxt = fire_in(g + 1, 1 - slot) if g + 1 < _G else []
        if g >= 2:
            for cp in outs[g - 2]:
                cp.wait()
        cur = []
        for k in range(_K):
            cp_in, r = in_flight[k]
            cp_in.wait()
            buf = bufs[slot * _K + k]
            row = rows[slot * _K + k]

            def inner(j, c, _buf=buf, _row=row, _r=r):
                gbase = j * (_LANES * _SHIFT_GRP)
                offs = [gbase + t * _LANES for t in range(_SHIFT_GRP)]
                vals = [_buf[pl.ds(_r + o, _LANES)] for o in offs]
                for o, v in zip(offs, vals):
                    _row[pl.ds(o, _LANES)] = v
                return c

            lax.fori_loop(0, _CHUNKS // _SHIFT_GRP, inner, 0)
            cur.append(pltpu.async_copy(row,
                                        press_hbm.at[base + g * _K + k],
                                        sem_out))
        outs.append(cur)
        in_flight = nxt
    for cp in outs[-2]:
        cp.wait()
    for cp in outs[-1]:
        cp.wait()


def _time_body(starts_ref, out_ref):
    iota = lax.broadcasted_iota(jnp.int32, out_ref.shape, 1).astype(
        jnp.float32)
    out_ref[...] = (starts_ref[...] + iota) * _INV_LEN


_TIME_BLK = 128


@jax.jit
def _sampler_call(sound_fragment, start_points):
    mesh = plsc.VectorSubcoreMesh(core_axis_name="c", subcore_axis_name="s")
    press_fn = pl.kernel(
        _press_body,
        mesh=mesh,
        out_type=jax.ShapeDtypeStruct((_BATCH_SIZE, _WINDOW_SIZE),
                                      jnp.float32),
        scratch_types=[
            pltpu.VMEM((_B_PER_W + _LANES,), jnp.int32),
        ] + [pltpu.VMEM((_BUF_W,), jnp.float32) for _ in range(2 * _K)]
        + [pltpu.VMEM((_WINDOW_SIZE,), jnp.float32) for _ in range(2 * _K)]
        + [
            pltpu.SemaphoreType.DMA,
            pltpu.SemaphoreType.DMA,
        ],
    )
    pressure = press_fn(sound_fragment, start_points)

    starts_f = start_points.astype(jnp.float32)[:, None]
    time_points = pl.pallas_call(
        _time_body,
        grid=(_BATCH_SIZE // _TIME_BLK,),
        in_specs=[pl.BlockSpec((_TIME_BLK, 1), lambda i: (i, 0))],
        out_specs=pl.BlockSpec((_TIME_BLK, _WINDOW_SIZE), lambda i: (i, 0)),
        out_shape=jax.ShapeDtypeStruct((_BATCH_SIZE, _WINDOW_SIZE),
                                       jnp.float32),
    )(starts_f)
    return time_points, pressure


def kernel(sound_fragment, key):
    start_points = jax.random.uniform(
        key, shape=(_BATCH_SIZE,), minval=0,
        maxval=_FRAGMENT_LENGTH - _WINDOW_SIZE,
    )
    start_points = jnp.floor(start_points).astype(jnp.int32)
    time_points, pressure_values = _sampler_call(sound_fragment, start_points)
    return (time_points, pressure_values)


# speedup vs baseline: 88.7534x; 1.0012x over previous
import functools
import jax
import jax.numpy as jnp
from jax import lax
from jax.experimental import pallas as pl
from jax.experimental.pallas import tpu as pltpu
from jax.experimental.pallas import tpu_sc as plsc

_FRAGMENT_LENGTH = 2097152
_WINDOW_SIZE = 4096
_BATCH_SIZE = 1024
_NUM_CORES = 2
_NUM_SUBCORES = 16
_NUM_WORKERS = _NUM_CORES * _NUM_SUBCORES
_B_PER_W = _BATCH_SIZE // _NUM_WORKERS
_LANES = 16
_CHUNKS = _WINDOW_SIZE // _LANES
_INV_LEN = 1.0 / _FRAGMENT_LENGTH
_K = 4
_G = _B_PER_W // _K
_BUF_W = _WINDOW_SIZE + 8
_SHIFT_GRP = 16


def _press_body(frag_hbm, starts_hbm, press_hbm, starts_v, *scratch):
    bufs = scratch[:2 * _K]
    rows = scratch[2 * _K:4 * _K]
    sem_in, sem_out = scratch[4 * _K], scratch[4 * _K + 1]
    wid = lax.axis_index("c") * _NUM_SUBCORES + lax.axis_index("s")
    base = wid * _B_PER_W
    pltpu.sync_copy(starts_hbm.at[pl.ds(base, _B_PER_W)],
                    starts_v.at[pl.ds(0, _B_PER_W)])

    def fire_in(g, slot):
        cps = []
        for k in range(_K):
            w = g * _K + k
            s = starts_v[pl.ds(w, _LANES)][0]
            q = pl.multiple_of((s // 8) * 8, 8)
            r = s - q
            cp = pltpu.async_copy(frag_hbm.at[pl.ds(q, _BUF_W)],
                                  bufs[slot * _K + k], sem_in)
            cps.append((cp, r))
        return cps

    in_flight = fire_in(0, 0)
    outs = []
    for g in range(_G):
        slot = g % 2
        nxt = fire_in(g + 1, 1 - slot) if g + 1 < _G else []
        if g >= 2:
            for cp in outs[g - 2]:
                cp.wait()
        cur = []
        for k in range(_K):
            cp_in, r = in_flight[k]
            cp_in.wait()
            buf = bufs[slot * _K + k]
            row = rows[slot * _K + k]

            def inner(j, c, _buf=buf, _row=row, _r=r):
                gbase = j * (_LANES * _SHIFT_GRP)
                offs = [gbase + t * _LANES for t in range(_SHIFT_GRP)]
                vals = [_buf[pl.ds(_r + o, _LANES)] for o in offs]
                for o, v in zip(offs, vals):
                    _row[pl.ds(o, _LANES)] = v
                return c

            lax.fori_loop(0, _CHUNKS // _SHIFT_GRP, inner, 0)
            cur.append(pltpu.async_copy(row,
                                        press_hbm.at[base + g * _K + k],
                                        sem_out))
        outs.append(cur)
        in_flight = nxt
    for cp in outs[-2]:
        cp.wait()
    for cp in outs[-1]:
        cp.wait()


def _time_body(starts_ref, out_ref):
    iota = lax.broadcasted_iota(jnp.int32, out_ref.shape, 1).astype(
        jnp.float32)
    starts_f = starts_ref[...].astype(jnp.float32)
    out_ref[...] = (starts_f + iota) * _INV_LEN


_TIME_BLK = 128


@jax.jit
def _sampler_call(sound_fragment, start_points):
    mesh = plsc.VectorSubcoreMesh(core_axis_name="c", subcore_axis_name="s")
    press_fn = pl.kernel(
        _press_body,
        mesh=mesh,
        out_type=jax.ShapeDtypeStruct((_BATCH_SIZE, _WINDOW_SIZE),
                                      jnp.float32),
        scratch_types=[
            pltpu.VMEM((_B_PER_W + _LANES,), jnp.int32),
        ] + [pltpu.VMEM((_BUF_W,), jnp.float32) for _ in range(2 * _K)]
        + [pltpu.VMEM((_WINDOW_SIZE,), jnp.float32) for _ in range(2 * _K)]
        + [
            pltpu.SemaphoreType.DMA,
            pltpu.SemaphoreType.DMA,
        ],
    )
    pressure = press_fn(sound_fragment, start_points)

    time_points = pl.pallas_call(
        _time_body,
        grid=(_BATCH_SIZE // _TIME_BLK,),
        in_specs=[pl.BlockSpec((_TIME_BLK, 1), lambda i: (i, 0))],
        out_specs=pl.BlockSpec((_TIME_BLK, _WINDOW_SIZE), lambda i: (i, 0)),
        out_shape=jax.ShapeDtypeStruct((_BATCH_SIZE, _WINDOW_SIZE),
                                       jnp.float32),
    )(start_points[:, None])
    return time_points, pressure


def kernel(sound_fragment, key):
    start_points = jax.random.uniform(
        key, shape=(_BATCH_SIZE,), minval=0,
        maxval=_FRAGMENT_LENGTH - _WINDOW_SIZE,
    )
    start_points = jnp.floor(start_points).astype(jnp.int32)
    time_points, pressure_values = _sampler_call(sound_fragment, start_points)
    return (time_points, pressure_values)


# speedup vs baseline: 92.8009x; 1.0456x over previous
import functools
import jax
import jax.numpy as jnp
from jax import lax
from jax.experimental import pallas as pl
from jax.experimental.pallas import tpu as pltpu
from jax.experimental.pallas import tpu_sc as plsc

_FRAGMENT_LENGTH = 2097152
_WINDOW_SIZE = 4096
_BATCH_SIZE = 1024
_NUM_CORES = 2
_NUM_SUBCORES = 16
_NUM_WORKERS = _NUM_CORES * _NUM_SUBCORES
_B_PER_W = _BATCH_SIZE // _NUM_WORKERS
_LANES = 16
_CHUNKS = _WINDOW_SIZE // _LANES
_INV_LEN = 1.0 / _FRAGMENT_LENGTH
_K = 4
_G = _B_PER_W // _K
_BUF_W = _WINDOW_SIZE + 8
_SHIFT_GRP = 16


def _press_body(frag_hbm, starts_hbm, press_hbm, starts_v, *scratch):
    bufs = scratch[:2 * _K]
    rows = scratch[2 * _K:4 * _K]
    sems_in = scratch[4 * _K:4 * _K + 2]
    sems_out = scratch[4 * _K + 2:4 * _K + 4]
    wid = lax.axis_index("c") * _NUM_SUBCORES + lax.axis_index("s")
    base = wid * _B_PER_W
    pltpu.sync_copy(starts_hbm.at[pl.ds(base, _B_PER_W)],
                    starts_v.at[pl.ds(0, _B_PER_W)])

    def start_of(w):
        s = starts_v[pl.ds(w, _LANES)][0]
        q = pl.multiple_of((s // 8) * 8, 8)
        return s, q, s - q

    def fire_in(g, slot):
        for k in range(_K):
            _, q, _ = start_of(g * _K + k)
            pltpu.async_copy(frag_hbm.at[pl.ds(q, _BUF_W)],
                             bufs[slot * _K + k], sems_in[slot])

    def drain_in(slot):
        for k in range(_K):
            pltpu.make_async_copy(frag_hbm.at[pl.ds(0, _BUF_W)],
                                  bufs[slot * _K + k], sems_in[slot]).wait()

    def drain_out(slot):
        for k in range(_K):
            pltpu.make_async_copy(rows[slot * _K + k], press_hbm.at[0],
                                  sems_out[slot]).wait()

    fire_in(0, 0)
    fire_in(1, 1)

    def body(i, c):
        for goff in range(2):
            slot = goff
            g = 2 * i + goff

            @pl.when(i > 0)
            def _():
                drain_out(slot)

            drain_in(slot)
            for k in range(_K):
                _, _, r = start_of(g * _K + k)
                buf = bufs[slot * _K + k]
                row = rows[slot * _K + k]

                def inner(j, c2, _buf=buf, _row=row, _r=r):
                    gbase = j * (_LANES * _SHIFT_GRP)
                    offs = [gbase + t * _LANES for t in range(_SHIFT_GRP)]
                    vals = [_buf[pl.ds(_r + o, _LANES)] for o in offs]
                    for o, v in zip(offs, vals):
                        _row[pl.ds(o, _LANES)] = v
                    return c2

                lax.fori_loop(0, _CHUNKS // _SHIFT_GRP, inner, 0)
                pltpu.async_copy(row, press_hbm.at[base + g * _K + k],
                                 sems_out[slot])

            @pl.when(g + 2 < _G)
            def _():
                fire_in(g + 2, slot)

        return c

    lax.fori_loop(0, _G // 2, body, 0)
    drain_out(0)
    drain_out(1)


def _time_body(starts_ref, out_ref):
    iota = lax.broadcasted_iota(jnp.int32, out_ref.shape, 1).astype(
        jnp.float32)
    starts_f = starts_ref[...].astype(jnp.float32)
    out_ref[...] = (starts_f + iota) * _INV_LEN


_TIME_BLK = 128


@jax.jit
def _sampler_call(sound_fragment, start_points):
    mesh = plsc.VectorSubcoreMesh(core_axis_name="c", subcore_axis_name="s")
    press_fn = pl.kernel(
        _press_body,
        mesh=mesh,
        out_type=jax.ShapeDtypeStruct((_BATCH_SIZE, _WINDOW_SIZE),
                                      jnp.float32),
        scratch_types=[
            pltpu.VMEM((_B_PER_W + _LANES,), jnp.int32),
        ] + [pltpu.VMEM((_BUF_W,), jnp.float32) for _ in range(2 * _K)]
        + [pltpu.VMEM((_WINDOW_SIZE,), jnp.float32) for _ in range(2 * _K)]
        + [
            pltpu.SemaphoreType.DMA,
            pltpu.SemaphoreType.DMA,
            pltpu.SemaphoreType.DMA,
            pltpu.SemaphoreType.DMA,
        ],
    )
    pressure = press_fn(sound_fragment, start_points)

    time_points = pl.pallas_call(
        _time_body,
        grid=(_BATCH_SIZE // _TIME_BLK,),
        in_specs=[pl.BlockSpec((_TIME_BLK, 1), lambda i: (i, 0))],
        out_specs=pl.BlockSpec((_TIME_BLK, _WINDOW_SIZE), lambda i: (i, 0)),
        out_shape=jax.ShapeDtypeStruct((_BATCH_SIZE, _WINDOW_SIZE),
                                       jnp.float32),
    )(start_points[:, None])
    return time_points, pressure


def kernel(sound_fragment, key):
    start_points = jax.random.uniform(
        key, shape=(_BATCH_SIZE,), minval=0,
        maxval=_FRAGMENT_LENGTH - _WINDOW_SIZE,
    )
    start_points = jnp.floor(start_points).astype(jnp.int32)
    time_points, pressure_values = _sampler_call(sound_fragment, start_points)
    return (time_points, pressure_values)


# speedup vs baseline: 96.5422x; 1.0403x over previous
import functools
import jax
import jax.numpy as jnp
from jax import lax
from jax.experimental import pallas as pl
from jax.experimental.pallas import tpu as pltpu
from jax.experimental.pallas import tpu_sc as plsc

_FRAGMENT_LENGTH = 2097152
_WINDOW_SIZE = 4096
_BATCH_SIZE = 1024
_MAXVAL = float(_FRAGMENT_LENGTH - _WINDOW_SIZE)
_NUM_CORES = 2
_NUM_SUBCORES = 16
_NUM_WORKERS = _NUM_CORES * _NUM_SUBCORES
_B_PER_W = _BATCH_SIZE // _NUM_WORKERS
_LANES = 16
_CHUNKS = _WINDOW_SIZE // _LANES
_INV_LEN = 1.0 / _FRAGMENT_LENGTH
_K = 4
_G = _B_PER_W // _K
_BUF_W = _WINDOW_SIZE + 8
_SHIFT_GRP = 16


def _threefry(k0, k1, x0, x1):
    ks2 = k0 ^ k1 ^ jnp.uint32(0x1BD11BDA)
    ks = (k0, k1, ks2)
    x0 = x0 + k0
    x1 = x1 + k1
    rotations = ((13, 15, 26, 6), (17, 29, 16, 24))
    for i in range(5):
        for r in rotations[i % 2]:
            x0 = x0 + x1
            x1 = (x1 << jnp.uint32(r)) | (x1 >> jnp.uint32(32 - r))
            x1 = x1 ^ x0
        x0 = x0 + ks[(i + 1) % 3]
        x1 = x1 + ks[(i + 2) % 3] + jnp.uint32(i + 1)
    return x0, x1


def _bits_to_start_f(bits):
    fb = (bits >> jnp.uint32(9)) | jnp.uint32(0x3F800000)
    u = lax.bitcast_convert_type(fb, jnp.float32) - jnp.float32(1.0)
    return jnp.maximum(jnp.float32(0.0), u * jnp.float32(_MAXVAL))


def _press_body(frag_hbm, key_hbm, press_hbm, key_v, starts_v, *scratch):
    bufs = scratch[:2 * _K]
    rows = scratch[2 * _K:4 * _K]
    sems_in = scratch[4 * _K:4 * _K + 2]
    sems_out = scratch[4 * _K + 2:4 * _K + 4]
    wid = lax.axis_index("c") * _NUM_SUBCORES + lax.axis_index("s")
    base = wid * _B_PER_W

    pltpu.sync_copy(key_hbm, key_v.at[pl.ds(0, 2)])
    k0 = key_v[pl.ds(0, _LANES)][0]
    k1 = key_v[pl.ds(1, _LANES)][0]
    zero = jnp.zeros((_LANES,), jnp.uint32)
    for c in range(_B_PER_W // _LANES):
        cnt = (lax.iota(jnp.int32, _LANES)
               + (base + c * _LANES)).astype(jnp.uint32)
        o0, o1 = _threefry(k0, k1, zero, cnt)
        sf = _bits_to_start_f(o0 ^ o1)
        starts_v[pl.ds(c * _LANES, _LANES)] = sf.astype(jnp.int32)

    def start_of(w):
        s = starts_v[pl.ds(w, _LANES)][0]
        q = pl.multiple_of((s // 8) * 8, 8)
        return s, q, s - q

    def fire_in(g, slot):
        for k in range(_K):
            _, q, _ = start_of(g * _K + k)
            pltpu.async_copy(frag_hbm.at[pl.ds(q, _BUF_W)],
                             bufs[slot * _K + k], sems_in[slot])

    def drain_in(slot):
        for k in range(_K):
            pltpu.make_async_copy(frag_hbm.at[pl.ds(0, _BUF_W)],
                                  bufs[slot * _K + k], sems_in[slot]).wait()

    def drain_out(slot):
        for k in range(_K):
            pltpu.make_async_copy(rows[slot * _K + k], press_hbm.at[0],
                                  sems_out[slot]).wait()

    fire_in(0, 0)
    fire_in(1, 1)

    def body(i, c):
        for goff in range(2):
            slot = goff
            g = 2 * i + goff

            @pl.when(i > 0)
            def _():
                drain_out(slot)

            drain_in(slot)
            for k in range(_K):
                _, _, r = start_of(g * _K + k)
                buf = bufs[slot * _K + k]
                row = rows[slot * _K + k]

                def inner(j, c2, _buf=buf, _row=row, _r=r):
                    gbase = j * (_LANES * _SHIFT_GRP)
                    offs = [gbase + t * _LANES for t in range(_SHIFT_GRP)]
                    vals = [_buf[pl.ds(_r + o, _LANES)] for o in offs]
                    for o, v in zip(offs, vals):
                        _row[pl.ds(o, _LANES)] = v
                    return c2

                lax.fori_loop(0, _CHUNKS // _SHIFT_GRP, inner, 0)
                pltpu.async_copy(row, press_hbm.at[base + g * _K + k],
                                 sems_out[slot])

            @pl.when(g + 2 < _G)
            def _():
                fire_in(g + 2, slot)

        return c

    lax.fori_loop(0, _G // 2, body, 0)
    drain_out(0)
    drain_out(1)


_TIME_BLK = 128


def _time_body(key_ref, out_ref):
    g = pl.program_id(0)
    k0 = key_ref[0]
    k1 = key_ref[1]
    nrow = _BATCH_SIZE // _TIME_BLK
    rows = (lax.broadcasted_iota(jnp.uint32, (nrow, _TIME_BLK), 0)
            * jnp.uint32(_TIME_BLK)
            + lax.broadcasted_iota(jnp.uint32, (nrow, _TIME_BLK), 1))
    o0, o1 = _threefry(k0, k1, jnp.zeros((nrow, _TIME_BLK), jnp.uint32),
                       rows)
    sf = jnp.floor(_bits_to_start_f(o0 ^ o1))
    st = jnp.transpose(sf)
    onehot = (lax.broadcasted_iota(jnp.int32, (_TIME_BLK, nrow), 1)
              == g)
    col = jnp.sum(jnp.where(onehot, st, jnp.float32(0.0)), axis=1,
                  keepdims=True)
    iota = lax.broadcasted_iota(jnp.int32, (_TIME_BLK, _WINDOW_SIZE),
                                1).astype(jnp.float32)
    out_ref[...] = (col + iota) * _INV_LEN


@jax.jit
def _sampler_call(sound_fragment, key_bits):
    mesh = plsc.VectorSubcoreMesh(core_axis_name="c", subcore_axis_name="s")
    press_fn = pl.kernel(
        _press_body,
        mesh=mesh,
        out_type=jax.ShapeDtypeStruct((_BATCH_SIZE, _WINDOW_SIZE),
                                      jnp.float32),
        scratch_types=[
            pltpu.VMEM((2 * _LANES,), jnp.uint32),
            pltpu.VMEM((_B_PER_W + _LANES,), jnp.int32),
        ] + [pltpu.VMEM((_BUF_W,), jnp.float32) for _ in range(2 * _K)]
        + [pltpu.VMEM((_WINDOW_SIZE,), jnp.float32) for _ in range(2 * _K)]
        + [
            pltpu.SemaphoreType.DMA,
            pltpu.SemaphoreType.DMA,
            pltpu.SemaphoreType.DMA,
            pltpu.SemaphoreType.DMA,
        ],
    )
    pressure = press_fn(sound_fragment, key_bits)

    time_points = pl.pallas_call(
        _time_body,
        grid=(_BATCH_SIZE // _TIME_BLK,),
        in_specs=[pl.BlockSpec(memory_space=pltpu.SMEM)],
        out_specs=pl.BlockSpec((_TIME_BLK, _WINDOW_SIZE), lambda i: (i, 0)),
        out_shape=jax.ShapeDtypeStruct((_BATCH_SIZE, _WINDOW_SIZE),
                                       jnp.float32),
    )(key_bits)
    return time_points, pressure


def kernel(sound_fragment, key):
    key_bits = jax.random.key_data(key)
    time_points, pressure_values = _sampler_call(sound_fragment, key_bits)
    return (time_points, pressure_values)
